# baseline (speedup 1.0000x reference)
import functools

import jax
import jax.numpy as jnp
import numpy as np
from jax import lax
from jax.experimental import pallas as pl
from jax.experimental.pallas import tpu as pltpu
from jax.experimental.pallas import tpu_sc as plsc

_THRESH = 0.7
_MIN_KEPT = 100000
_IGNORE = -1

_B = 8
_C = 19
_H = 512
_W = 512
_TH = 64
_N = _B * _H * _W

_NW = 32
_ROWS = 128
_HB = 32768
_NSTRIPE = 16
_SW = _HB // _NSTRIPE
_TARGET = _MIN_KEPT + 1
_THRESH_BITS = int(np.float32(_THRESH).view(np.int32))

_mesh = plsc.VectorSubcoreMesh(core_axis_name="c", subcore_axis_name="s")


def _lane(v, j):
    return jnp.max(jnp.where(lax.iota(jnp.int32, 16) == j, v, jnp.int32(0)))


def _pass1_body(pred_ref, tgt_ref, prob_ref, nll_ref, s07_ref, acc_ref):
    x = pred_ref[0]
    t = tgt_ref[0]
    tt = jnp.where(t == _IGNORE, 0, t)
    m = jnp.max(x, axis=0)
    e = jnp.exp(x - m[None])
    s = jnp.sum(e, axis=0)
    onehot = jax.lax.broadcasted_iota(jnp.int32, (_C, _TH, _W), 0) == tt[None]
    xt = jnp.sum(jnp.where(onehot, x, 0.0), axis=0)
    prob = jnp.exp(xt - m) / s
    nll = jnp.log(s) + m - xt
    prob_ref[0] = prob
    nll_ref[0] = nll

    step = pl.program_id(0) * (_H // _TH) + pl.program_id(1)

    @pl.when(step == 0)
    def _():
        acc_ref[0] = 0.0
        acc_ref[1] = 0.0

    keep = prob < _THRESH
    acc_ref[0] += jnp.sum(jnp.where(keep, nll, 0.0))
    acc_ref[1] += jnp.sum(keep.astype(jnp.float32))

    @pl.when(step == _B * (_H // _TH) - 1)
    def _():
        s07_ref[0, 0] = acc_ref[0]
        s07_ref[0, 1] = acc_ref[1]


def _hist_round(stage):
    out_type = (
        jax.ShapeDtypeStruct((_NSTRIPE, _NW, _SW), jnp.int32),
        jax.ShapeDtypeStruct((_NW, 16), jnp.int32),
    )
    scratch = [
        pltpu.VMEM((_ROWS // 2, _W), jnp.float32),
        pltpu.VMEM((_HB,), jnp.int32),
        pltpu.VMEM((_HB,), jnp.int32),
        pltpu.VMEM((16,), jnp.int32),
        pltpu.VMEM((16,), jnp.int32),
        pltpu.SemaphoreType.DMA,
    ]

    def body(prob_hbm, *rest):
        if stage == 1:
            (hist_out, coarse_out, buf, hist_v, hist2_v, coarse_v, pick_v,
             sem) = rest
        else:
            (pick_hbm, hist_out, coarse_out, buf, hist_v, hist2_v, coarse_v,
             pick_v, sem) = rest
        wid = lax.axis_index("s") * 2 + lax.axis_index("c")
        bq = lax.shift_right_logical(wid, 2)
        hq = jnp.bitwise_and(wid, 3) * _ROWS
        if stage == 2:
            pltpu.sync_copy(pick_hbm, pick_v)
            b1 = _lane(pick_v[...], 0)

        zeros = jnp.zeros((16,), jnp.int32)

        def zero_body(i, _):
            hist_v[pl.ds(pl.multiple_of(i * 16, 16), 16)] = zeros
            hist2_v[pl.ds(pl.multiple_of(i * 16, 16), 16)] = zeros
            return 0

        lax.fori_loop(0, _HB // 16, zero_body, 0, unroll=8)

        ones = jnp.ones((16,), jnp.int32)

        def row_body(r, _):
            for j in range(_W // 16):
                tgt_hist = hist_v if j % 2 == 0 else hist2_v
                v = buf[r, pl.ds(j * 16, 16)]
                bits = lax.bitcast_convert_type(v, jnp.int32)
                if stage == 1:
                    plsc.addupdate_scatter(
                        tgt_hist, [lax.shift_right_logical(bits, 15)], ones)
                else:
                    match = lax.shift_right_logical(bits, 15) == b1
                    plsc.addupdate_scatter(
                        tgt_hist, [jnp.bitwise_and(bits, _HB - 1)], ones,
                        mask=match)
            return 0

        for half in range(2):
            pltpu.sync_copy(
                prob_hbm.at[bq, pl.ds(hq + half * (_ROWS // 2), _ROWS // 2)],
                buf)
            lax.fori_loop(0, _ROWS // 2, row_body, 0)

        coarse = zeros
        lanes = lax.iota(jnp.int32, 16)
        for s in range(_NSTRIPE):
            def sum_body(j, a, s=s):
                ix = pl.ds(pl.multiple_of(s * _SW + j * 16, 16), 16)
                h = hist_v[ix] + hist2_v[ix]
                hist_v[ix] = h
                return a + h
            acc = lax.fori_loop(0, _SW // 16, sum_body, zeros, unroll=8)
            coarse = jnp.where(lanes == s, jnp.sum(acc), coarse)
        coarse_v[...] = coarse

        copies = [
            pltpu.async_copy(
                hist_v.at[pl.ds(s * _SW, _SW)], hist_out.at[s, wid], sem)
            for s in range(_NSTRIPE)
        ]
        for c in copies:
            c.wait()
        pltpu.sync_copy(coarse_v, coarse_out.at[wid])

    return functools.partial(
        pl.kernel, out_type=out_type, mesh=_mesh, scratch_types=scratch,
        compiler_params=pltpu.CompilerParams(
            needs_layout_passes=False))(body)


_sc_hist1 = _hist_round(1)
_sc_hist2 = _hist_round(2)


def _pick_round(stage):
    scratch = [
        pltpu.VMEM((_NW, _SW), jnp.int32),
        pltpu.VMEM((_NW, 16), jnp.int32),
        pltpu.VMEM((16,), jnp.int32),
        pltpu.VMEM((16,), jnp.int32),
    ]

    def body(hist_hbm, coarse_hbm, *rest):
        if stage == 1:
            (out_hbm, sbuf, cbuf, ovec, pick_v) = rest
        else:
            (pick_hbm, out_hbm, sbuf, cbuf, ovec, pick_v) = rest
        sid = lax.axis_index("s")
        pltpu.sync_copy(coarse_hbm, cbuf)
        if stage == 1:
            target = jnp.int32(_TARGET)
        else:
            pltpu.sync_copy(pick_hbm, pick_v)
            b1 = _lane(pick_v[...], 0)
            target = _lane(pick_v[...], 1)

        tot = jnp.zeros((16,), jnp.int32)
        for r in range(_NW):
            tot = tot + cbuf[r]
        cum = plsc.cumsum(tot)
        sstar = jnp.max(plsc.all_reduce_ffs(cum >= target))
        before_stripe = jnp.max(jnp.where(cum < target, cum, jnp.int32(0)))
        krem = target - before_stripe

        pltpu.sync_copy(hist_hbm.at[sid], sbuf)

        def chunk(c, carry):
            bin_, before, run = carry
            acc = jnp.zeros((16,), jnp.int32)
            for r in range(_NW):
                acc = acc + sbuf[r, pl.ds(pl.multiple_of(c * 16, 16), 16)]
            cs = plsc.cumsum(acc) + run
            ge = cs >= krem
            has = jnp.max(ge.astype(jnp.int32)) > 0
            idx = jnp.max(plsc.all_reduce_ffs(ge))
            fresh = (bin_ < 0) & has
            newbin = jnp.where(fresh, c * 16 + idx, bin_)
            newbefore = jnp.where(
                fresh, jnp.max(jnp.where(cs < krem, cs, run)), before)
            return (newbin, newbefore, jnp.max(cs))

        bin_, before, _ = lax.fori_loop(
            0, _SW // 16, chunk, (jnp.int32(-1), jnp.int32(0), jnp.int32(0)))

        lanes = lax.iota(jnp.int32, 16)
        if stage == 1:
            bg = sid * _SW + bin_
            krem2 = krem - before
            pred = (bg < jnp.int32(_THRESH_BITS >> 15)).astype(jnp.int32)
            vec = jnp.where(lanes == 0, bg,
                            jnp.where(lanes == 1, krem2,
                                      jnp.where(lanes == 2, pred,
                                                jnp.int32(0))))
        else:
            t = jnp.bitwise_or(lax.shift_left(b1, 15), sid * _SW + bin_)
            vec = jnp.zeros((16,), jnp.int32) + jnp.maximum(
                t, jnp.int32(_THRESH_BITS))
        ovec[...] = vec

        @pl.when(sid == sstar)
        def _():
            pltpu.sync_copy(ovec, out_hbm)

    return functools.partial(
        pl.kernel, out_type=jax.ShapeDtypeStruct((16,), jnp.int32),
        mesh=_mesh, scratch_types=scratch,
        compiler_params=pltpu.CompilerParams(
            needs_layout_passes=False))(body)


_sc_pick1 = _pick_round(1)
_sc_pick2 = _pick_round(2)


def _final_body(thr_ref, prob_ref, nll_ref, out_ref):
    thr = jax.lax.bitcast_convert_type(thr_ref[0, 0], jnp.float32)
    prob = prob_ref[...]
    keep = prob < thr
    s = jnp.sum(jnp.where(keep, nll_ref[...], 0.0))
    c = jnp.sum(keep.astype(jnp.float32))
    out_ref[0, 0] = s / c


def kernel(predict, target):
    tgt = target.astype(jnp.int32)

    prob, nll, s07 = pl.pallas_call(
        _pass1_body,
        grid=(_B, _H // _TH),
        in_specs=[
            pl.BlockSpec((1, _C, _TH, _W), lambda b, h: (b, 0, h, 0)),
            pl.BlockSpec((1, _TH, _W), lambda b, h: (b, h, 0)),
        ],
        out_specs=[
            pl.BlockSpec((1, _TH, _W), lambda b, h: (b, h, 0)),
            pl.BlockSpec((1, _TH, _W), lambda b, h: (b, h, 0)),
            pl.BlockSpec(memory_space=pltpu.SMEM),
        ],
        out_shape=[
            jax.ShapeDtypeStruct((_B, _H, _W), jnp.float32),
            jax.ShapeDtypeStruct((_B, _H, _W), jnp.float32),
            jax.ShapeDtypeStruct((1, 2), jnp.float32),
        ],
        scratch_shapes=[pltpu.SMEM((2,), jnp.float32)],
    )(predict, tgt)

    hist1, coarse1 = _sc_hist1(prob)
    pick1 = _sc_pick1(hist1, coarse1)

    def _common(_):
        return s07[0, 0] / s07[0, 1]

    def _rare(_):
        hist2, coarse2 = _sc_hist2(prob, pick1)
        thr_vec = _sc_pick2(hist2, coarse2, pick1)
        loss = pl.pallas_call(
            _final_body,
            in_specs=[
                pl.BlockSpec(memory_space=pltpu.SMEM),
                pl.BlockSpec((_B, _H, _W), lambda: (0, 0, 0)),
                pl.BlockSpec((_B, _H, _W), lambda: (0, 0, 0)),
            ],
            out_specs=pl.BlockSpec(memory_space=pltpu.SMEM),
            out_shape=jax.ShapeDtypeStruct((1, 1), jnp.float32),
        )(thr_vec[0:1].reshape(1, 1), prob, nll)
        return loss[0, 0]

    return lax.cond(pick1[2] == 1, _common, _rare, None)


# speedup vs baseline: 1.1548x; 1.1548x over previous
import functools

import jax
import jax.numpy as jnp
import numpy as np
from jax import lax
from jax.experimental import pallas as pl
from jax.experimental.pallas import tpu as pltpu
from jax.experimental.pallas import tpu_sc as plsc

_THRESH = 0.7
_MIN_KEPT = 100000
_IGNORE = -1

_B = 8
_C = 19
_H = 512
_W = 512
_TH = 128
_N = _B * _H * _W

_NW = 32
_ROWS = 128
_HB = 32768
_NSTRIPE = 16
_SW = _HB // _NSTRIPE
_TARGET = _MIN_KEPT + 1
_THRESH_BITS = int(np.float32(_THRESH).view(np.int32))

_mesh = plsc.VectorSubcoreMesh(core_axis_name="c", subcore_axis_name="s")


def _lane(v, j):
    return jnp.max(jnp.where(lax.iota(jnp.int32, 16) == j, v, jnp.int32(0)))


def _pass1_body(pred_ref, tgt_ref, prob_ref, nll_ref, s07_ref, acc_ref):
    x = pred_ref[0]
    t = tgt_ref[0]
    tt = jnp.where(t == _IGNORE, 0, t)
    m = jnp.max(x, axis=0)
    e = jnp.exp(x - m[None])
    s = jnp.sum(e, axis=0)
    onehot = jax.lax.broadcasted_iota(jnp.int32, (_C, _TH, _W), 0) == tt[None]
    xt = jnp.sum(jnp.where(onehot, x, 0.0), axis=0)
    prob = jnp.exp(xt - m) / s
    nll = jnp.log(s) + m - xt
    prob_ref[0] = prob
    nll_ref[0] = nll

    step = pl.program_id(0) * (_H // _TH) + pl.program_id(1)

    @pl.when(step == 0)
    def _():
        acc_ref[0] = 0.0
        acc_ref[1] = 0.0

    keep = prob < _THRESH
    acc_ref[0] += jnp.sum(jnp.where(keep, nll, 0.0))
    acc_ref[1] += jnp.sum(keep.astype(jnp.float32))

    @pl.when(step == _B * (_H // _TH) - 1)
    def _():
        s07_ref[0, 0] = acc_ref[0]
        s07_ref[0, 1] = acc_ref[1]


def _hist_round(stage):
    out_type = (
        jax.ShapeDtypeStruct((_NSTRIPE, _NW, _SW), jnp.int32),
        jax.ShapeDtypeStruct((_NW, 16), jnp.int32),
    )
    scratch = [
        pltpu.VMEM((_ROWS, _W), jnp.float32),
        pltpu.VMEM((_HB,), jnp.int32),
        pltpu.VMEM((16,), jnp.int32),
        pltpu.VMEM((16,), jnp.int32),
        pltpu.SemaphoreType.DMA,
    ]

    def body(prob_hbm, *rest):
        if stage == 1:
            (hist_out, coarse_out, buf, hist_v, coarse_v, pick_v, sem) = rest
        else:
            (pick_hbm, hist_out, coarse_out, buf, hist_v, coarse_v,
             pick_v, sem) = rest
        wid = lax.axis_index("s") * 2 + lax.axis_index("c")
        bq = lax.shift_right_logical(wid, 2)
        hq = jnp.bitwise_and(wid, 3) * _ROWS
        pltpu.sync_copy(prob_hbm.at[bq, pl.ds(hq, _ROWS)], buf)
        if stage == 2:
            pltpu.sync_copy(pick_hbm, pick_v)
            b1 = _lane(pick_v[...], 0)

        zeros = jnp.zeros((16,), jnp.int32)

        def zero_body(i, _):
            hist_v[pl.ds(pl.multiple_of(i * 16, 16), 16)] = zeros
            return 0

        lax.fori_loop(0, _HB // 16, zero_body, 0, unroll=8)

        ones = jnp.ones((16,), jnp.int32)

        def row_body(r, _):
            for j in range(_W // 16):
                v = buf[r, pl.ds(j * 16, 16)]
                bits = lax.bitcast_convert_type(v, jnp.int32)
                if stage == 1:
                    plsc.addupdate_scatter(
                        hist_v, [lax.shift_right_logical(bits, 15)], ones)
                else:
                    match = lax.shift_right_logical(bits, 15) == b1
                    plsc.addupdate_scatter(
                        hist_v, [jnp.bitwise_and(bits, _HB - 1)], ones,
                        mask=match)
            return 0

        lax.fori_loop(0, _ROWS, row_body, 0)

        coarse = zeros
        lanes = lax.iota(jnp.int32, 16)
        for s in range(_NSTRIPE):
            def sum_body(j, a, s=s):
                return a + hist_v[pl.ds(pl.multiple_of(s * _SW + j * 16, 16),
                                        16)]
            acc = lax.fori_loop(0, _SW // 16, sum_body, zeros, unroll=8)
            coarse = jnp.where(lanes == s, jnp.sum(acc), coarse)
        coarse_v[...] = coarse

        copies = [
            pltpu.async_copy(
                hist_v.at[pl.ds(s * _SW, _SW)], hist_out.at[s, wid], sem)
            for s in range(_NSTRIPE)
        ]
        for c in copies:
            c.wait()
        pltpu.sync_copy(coarse_v, coarse_out.at[wid])

    return functools.partial(
        pl.kernel, out_type=out_type, mesh=_mesh, scratch_types=scratch,
        compiler_params=pltpu.CompilerParams(
            needs_layout_passes=False))(body)


_sc_hist1 = _hist_round(1)
_sc_hist2 = _hist_round(2)


def _pick_round(stage):
    scratch = [
        pltpu.VMEM((_NW, _SW), jnp.int32),
        pltpu.VMEM((_NW, 16), jnp.int32),
        pltpu.VMEM((16,), jnp.int32),
        pltpu.VMEM((16,), jnp.int32),
    ]

    def body(hist_hbm, coarse_hbm, *rest):
        if stage == 1:
            (out_hbm, sbuf, cbuf, ovec, pick_v) = rest
        else:
            (pick_hbm, out_hbm, sbuf, cbuf, ovec, pick_v) = rest
        sid = lax.axis_index("s")
        pltpu.sync_copy(coarse_hbm, cbuf)
        if stage == 1:
            target = jnp.int32(_TARGET)
        else:
            pltpu.sync_copy(pick_hbm, pick_v)
            b1 = _lane(pick_v[...], 0)
            target = _lane(pick_v[...], 1)

        tot = jnp.zeros((16,), jnp.int32)
        for r in range(_NW):
            tot = tot + cbuf[r]
        cum = plsc.cumsum(tot)
        sstar = jnp.max(plsc.all_reduce_ffs(cum >= target))
        before_stripe = jnp.max(jnp.where(cum < target, cum, jnp.int32(0)))
        krem = target - before_stripe

        pltpu.sync_copy(hist_hbm.at[sid], sbuf)

        def chunk(c, carry):
            bin_, before, run = carry
            acc = jnp.zeros((16,), jnp.int32)
            for r in range(_NW):
                acc = acc + sbuf[r, pl.ds(pl.multiple_of(c * 16, 16), 16)]
            cs = plsc.cumsum(acc) + run
            ge = cs >= krem
            has = jnp.max(ge.astype(jnp.int32)) > 0
            idx = jnp.max(plsc.all_reduce_ffs(ge))
            fresh = (bin_ < 0) & has
            newbin = jnp.where(fresh, c * 16 + idx, bin_)
            newbefore = jnp.where(
                fresh, jnp.max(jnp.where(cs < krem, cs, run)), before)
            return (newbin, newbefore, jnp.max(cs))

        bin_, before, _ = lax.fori_loop(
            0, _SW // 16, chunk, (jnp.int32(-1), jnp.int32(0), jnp.int32(0)))

        lanes = lax.iota(jnp.int32, 16)
        if stage == 1:
            bg = sid * _SW + bin_
            krem2 = krem - before
            pred = (bg < jnp.int32(_THRESH_BITS >> 15)).astype(jnp.int32)
            vec = jnp.where(lanes == 0, bg,
                            jnp.where(lanes == 1, krem2,
                                      jnp.where(lanes == 2, pred,
                                                jnp.int32(0))))
        else:
            t = jnp.bitwise_or(lax.shift_left(b1, 15), sid * _SW + bin_)
            vec = jnp.zeros((16,), jnp.int32) + jnp.maximum(
                t, jnp.int32(_THRESH_BITS))
        ovec[...] = vec

        @pl.when(sid == sstar)
        def _():
            pltpu.sync_copy(ovec, out_hbm)

    return functools.partial(
        pl.kernel, out_type=jax.ShapeDtypeStruct((16,), jnp.int32),
        mesh=_mesh, scratch_types=scratch,
        compiler_params=pltpu.CompilerParams(
            needs_layout_passes=False))(body)


_sc_pick1 = _pick_round(1)
_sc_pick2 = _pick_round(2)


def _final_body(thr_ref, prob_ref, nll_ref, out_ref):
    thr = jax.lax.bitcast_convert_type(thr_ref[0, 0], jnp.float32)
    prob = prob_ref[...]
    keep = prob < thr
    s = jnp.sum(jnp.where(keep, nll_ref[...], 0.0))
    c = jnp.sum(keep.astype(jnp.float32))
    out_ref[0, 0] = s / c


def kernel(predict, target):
    tgt = target.astype(jnp.int32)

    prob, nll, s07 = pl.pallas_call(
        _pass1_body,
        grid=(_B, _H // _TH),
        in_specs=[
            pl.BlockSpec((1, _C, _TH, _W), lambda b, h: (b, 0, h, 0)),
            pl.BlockSpec((1, _TH, _W), lambda b, h: (b, h, 0)),
        ],
        out_specs=[
            pl.BlockSpec((1, _TH, _W), lambda b, h: (b, h, 0)),
            pl.BlockSpec((1, _TH, _W), lambda b, h: (b, h, 0)),
            pl.BlockSpec(memory_space=pltpu.SMEM),
        ],
        out_shape=[
            jax.ShapeDtypeStruct((_B, _H, _W), jnp.float32),
            jax.ShapeDtypeStruct((_B, _H, _W), jnp.float32),
            jax.ShapeDtypeStruct((1, 2), jnp.float32),
        ],
        scratch_shapes=[pltpu.SMEM((2,), jnp.float32)],
    )(predict, tgt)

    hist1, coarse1 = _sc_hist1(prob)
    pick1 = _sc_pick1(hist1, coarse1)

    def _common(_):
        return s07[0, 0] / s07[0, 1]

    def _rare(_):
        hist2, coarse2 = _sc_hist2(prob, pick1)
        thr_vec = _sc_pick2(hist2, coarse2, pick1)
        loss = pl.pallas_call(
            _final_body,
            in_specs=[
                pl.BlockSpec(memory_space=pltpu.SMEM),
                pl.BlockSpec((_B, _H, _W), lambda: (0, 0, 0)),
                pl.BlockSpec((_B, _H, _W), lambda: (0, 0, 0)),
            ],
            out_specs=pl.BlockSpec(memory_space=pltpu.SMEM),
            out_shape=jax.ShapeDtypeStruct((1, 1), jnp.float32),
        )(thr_vec[0:1].reshape(1, 1), prob, nll)
        return loss[0, 0]

    return lax.cond(pick1[2] == 1, _common, _rare, None)


# speedup vs baseline: 1.1754x; 1.0178x over previous
import functools

import jax
import jax.numpy as jnp
import numpy as np
from jax import lax
from jax.experimental import pallas as pl
from jax.experimental.pallas import tpu as pltpu
from jax.experimental.pallas import tpu_sc as plsc

_THRESH = 0.7
_MIN_KEPT = 100000
_IGNORE = -1

_B = 8
_C = 19
_H = 512
_W = 512
_TH = 256
_N = _B * _H * _W

_NW = 32
_ROWS = 128
_HB = 32768
_NSTRIPE = 16
_SW = _HB // _NSTRIPE
_TARGET = _MIN_KEPT + 1
_THRESH_BITS = int(np.float32(_THRESH).view(np.int32))

_mesh = plsc.VectorSubcoreMesh(core_axis_name="c", subcore_axis_name="s")


def _lane(v, j):
    return jnp.max(jnp.where(lax.iota(jnp.int32, 16) == j, v, jnp.int32(0)))


def _pass1_body(pred_ref, tgt_ref, prob_ref, nll_ref, s07_ref, acc_ref):
    x = pred_ref[0]
    t = tgt_ref[0]
    tt = jnp.where(t == _IGNORE, 0, t)
    m = jnp.max(x, axis=0)
    e = jnp.exp(x - m[None])
    s = jnp.sum(e, axis=0)
    onehot = jax.lax.broadcasted_iota(jnp.int32, (_C, _TH, _W), 0) == tt[None]
    xt = jnp.sum(jnp.where(onehot, x, 0.0), axis=0)
    prob = jnp.exp(xt - m) / s
    nll = jnp.log(s) + m - xt
    prob_ref[0] = prob
    nll_ref[0] = nll

    step = pl.program_id(0) * (_H // _TH) + pl.program_id(1)

    @pl.when(step == 0)
    def _():
        acc_ref[0] = 0.0
        acc_ref[1] = 0.0

    keep = prob < _THRESH
    acc_ref[0] += jnp.sum(jnp.where(keep, nll, 0.0))
    acc_ref[1] += jnp.sum(keep.astype(jnp.float32))

    @pl.when(step == _B * (_H // _TH) - 1)
    def _():
        s07_ref[0, 0] = acc_ref[0]
        s07_ref[0, 1] = acc_ref[1]


def _hist_round(stage):
    out_type = (
        jax.ShapeDtypeStruct((_NSTRIPE, _NW, _SW), jnp.int32),
        jax.ShapeDtypeStruct((_NW, 16), jnp.int32),
    )
    scratch = [
        pltpu.VMEM((_ROWS, _W), jnp.float32),
        pltpu.VMEM((_HB,), jnp.int32),
        pltpu.VMEM((16,), jnp.int32),
        pltpu.VMEM((16,), jnp.int32),
        pltpu.SemaphoreType.DMA,
    ]

    def body(prob_hbm, *rest):
        if stage == 1:
            (hist_out, coarse_out, buf, hist_v, coarse_v, pick_v, sem) = rest
        else:
            (pick_hbm, hist_out, coarse_out, buf, hist_v, coarse_v,
             pick_v, sem) = rest
        wid = lax.axis_index("s") * 2 + lax.axis_index("c")
        bq = lax.shift_right_logical(wid, 2)
        hq = jnp.bitwise_and(wid, 3) * _ROWS
        pltpu.sync_copy(prob_hbm.at[bq, pl.ds(hq, _ROWS)], buf)
        if stage == 2:
            pltpu.sync_copy(pick_hbm, pick_v)
            b1 = _lane(pick_v[...], 0)

        zeros = jnp.zeros((16,), jnp.int32)

        def zero_body(i, _):
            hist_v[pl.ds(pl.multiple_of(i * 16, 16), 16)] = zeros
            return 0

        lax.fori_loop(0, _HB // 16, zero_body, 0, unroll=8)

        ones = jnp.ones((16,), jnp.int32)

        def row_body(r, _):
            for j in range(_W // 16):
                v = buf[r, pl.ds(j * 16, 16)]
                bits = lax.bitcast_convert_type(v, jnp.int32)
                if stage == 1:
                    plsc.addupdate_scatter(
                        hist_v, [lax.shift_right_logical(bits, 15)], ones)
                else:
                    match = lax.shift_right_logical(bits, 15) == b1
                    plsc.addupdate_scatter(
                        hist_v, [jnp.bitwise_and(bits, _HB - 1)], ones,
                        mask=match)
            return 0

        lax.fori_loop(0, _ROWS, row_body, 0)

        coarse = zeros
        lanes = lax.iota(jnp.int32, 16)
        for s in range(_NSTRIPE):
            def sum_body(j, a, s=s):
                return a + hist_v[pl.ds(pl.multiple_of(s * _SW + j * 16, 16),
                                        16)]
            acc = lax.fori_loop(0, _SW // 16, sum_body, zeros, unroll=8)
            coarse = jnp.where(lanes == s, jnp.sum(acc), coarse)
        coarse_v[...] = coarse

        copies = [
            pltpu.async_copy(
                hist_v.at[pl.ds(s * _SW, _SW)], hist_out.at[s, wid], sem)
            for s in range(_NSTRIPE)
        ]
        for c in copies:
            c.wait()
        pltpu.sync_copy(coarse_v, coarse_out.at[wid])

    return functools.partial(
        pl.kernel, out_type=out_type, mesh=_mesh, scratch_types=scratch,
        compiler_params=pltpu.CompilerParams(
            needs_layout_passes=False))(body)


_sc_hist1 = _hist_round(1)
_sc_hist2 = _hist_round(2)


def _pick_round(stage):
    scratch = [
        pltpu.VMEM((_NW, _SW), jnp.int32),
        pltpu.VMEM((_NW, 16), jnp.int32),
        pltpu.VMEM((16,), jnp.int32),
        pltpu.VMEM((16,), jnp.int32),
    ]

    def body(hist_hbm, coarse_hbm, *rest):
        if stage == 1:
            (out_hbm, sbuf, cbuf, ovec, pick_v) = rest
        else:
            (pick_hbm, out_hbm, sbuf, cbuf, ovec, pick_v) = rest
        sid = lax.axis_index("s")
        pltpu.sync_copy(coarse_hbm, cbuf)
        if stage == 1:
            target = jnp.int32(_TARGET)
        else:
            pltpu.sync_copy(pick_hbm, pick_v)
            b1 = _lane(pick_v[...], 0)
            target = _lane(pick_v[...], 1)

        tot = jnp.zeros((16,), jnp.int32)
        for r in range(_NW):
            tot = tot + cbuf[r]
        cum = plsc.cumsum(tot)
        sstar = jnp.max(plsc.all_reduce_ffs(cum >= target))
        before_stripe = jnp.max(jnp.where(cum < target, cum, jnp.int32(0)))
        krem = target - before_stripe

        pltpu.sync_copy(hist_hbm.at[sid], sbuf)

        def chunk(c, carry):
            bin_, before, run = carry
            acc = jnp.zeros((16,), jnp.int32)
            for r in range(_NW):
                acc = acc + sbuf[r, pl.ds(pl.multiple_of(c * 16, 16), 16)]
            cs = plsc.cumsum(acc) + run
            ge = cs >= krem
            has = jnp.max(ge.astype(jnp.int32)) > 0
            idx = jnp.max(plsc.all_reduce_ffs(ge))
            fresh = (bin_ < 0) & has
            newbin = jnp.where(fresh, c * 16 + idx, bin_)
            newbefore = jnp.where(
                fresh, jnp.max(jnp.where(cs < krem, cs, run)), before)
            return (newbin, newbefore, jnp.max(cs))

        bin_, before, _ = lax.fori_loop(
            0, _SW // 16, chunk, (jnp.int32(-1), jnp.int32(0), jnp.int32(0)))

        lanes = lax.iota(jnp.int32, 16)
        if stage == 1:
            bg = sid * _SW + bin_
            krem2 = krem - before
            pred = (bg < jnp.int32(_THRESH_BITS >> 15)).astype(jnp.int32)
            vec = jnp.where(lanes == 0, bg,
                            jnp.where(lanes == 1, krem2,
                                      jnp.where(lanes == 2, pred,
                                                jnp.int32(0))))
        else:
            t = jnp.bitwise_or(lax.shift_left(b1, 15), sid * _SW + bin_)
            vec = jnp.zeros((16,), jnp.int32) + jnp.maximum(
                t, jnp.int32(_THRESH_BITS))
        ovec[...] = vec

        @pl.when(sid == sstar)
        def _():
            pltpu.sync_copy(ovec, out_hbm)

    return functools.partial(
        pl.kernel, out_type=jax.ShapeDtypeStruct((16,), jnp.int32),
        mesh=_mesh, scratch_types=scratch,
        compiler_params=pltpu.CompilerParams(
            needs_layout_passes=False))(body)


_sc_pick1 = _pick_round(1)
_sc_pick2 = _pick_round(2)


def _final_body(thr_ref, prob_ref, nll_ref, out_ref):
    thr = jax.lax.bitcast_convert_type(thr_ref[0, 0], jnp.float32)
    prob = prob_ref[...]
    keep = prob < thr
    s = jnp.sum(jnp.where(keep, nll_ref[...], 0.0))
    c = jnp.sum(keep.astype(jnp.float32))
    out_ref[0, 0] = s / c


def kernel(predict, target):
    tgt = target.astype(jnp.int32)

    prob, nll, s07 = pl.pallas_call(
        _pass1_body,
        grid=(_B, _H // _TH),
        in_specs=[
            pl.BlockSpec((1, _C, _TH, _W), lambda b, h: (b, 0, h, 0)),
            pl.BlockSpec((1, _TH, _W), lambda b, h: (b, h, 0)),
        ],
        out_specs=[
            pl.BlockSpec((1, _TH, _W), lambda b, h: (b, h, 0)),
            pl.BlockSpec((1, _TH, _W), lambda b, h: (b, h, 0)),
            pl.BlockSpec(memory_space=pltpu.SMEM),
        ],
        out_shape=[
            jax.ShapeDtypeStruct((_B, _H, _W), jnp.float32),
            jax.ShapeDtypeStruct((_B, _H, _W), jnp.float32),
            jax.ShapeDtypeStruct((1, 2), jnp.float32),
        ],
        scratch_shapes=[pltpu.SMEM((2,), jnp.float32)],
    )(predict, tgt)

    hist1, coarse1 = _sc_hist1(prob)
    pick1 = _sc_pick1(hist1, coarse1)

    def _common(_):
        return s07[0, 0] / s07[0, 1]

    def _rare(_):
        hist2, coarse2 = _sc_hist2(prob, pick1)
        thr_vec = _sc_pick2(hist2, coarse2, pick1)
        loss = pl.pallas_call(
            _final_body,
            in_specs=[
                pl.BlockSpec(memory_space=pltpu.SMEM),
                pl.BlockSpec((_B, _H, _W), lambda: (0, 0, 0)),
                pl.BlockSpec((_B, _H, _W), lambda: (0, 0, 0)),
            ],
            out_specs=pl.BlockSpec(memory_space=pltpu.SMEM),
            out_shape=jax.ShapeDtypeStruct((1, 1), jnp.float32),
        )(thr_vec[0:1].reshape(1, 1), prob, nll)
        return loss[0, 0]

    return lax.cond(pick1[2] == 1, _common, _rare, None)


# speedup vs baseline: 1.1861x; 1.0091x over previous
import functools

import jax
import jax.numpy as jnp
import numpy as np
from jax import lax
from jax.experimental import pallas as pl
from jax.experimental.pallas import tpu as pltpu
from jax.experimental.pallas import tpu_sc as plsc

_THRESH = 0.7
_MIN_KEPT = 100000
_IGNORE = -1

_B = 8
_C = 19
_H = 512
_W = 512
_TH = 256
_N = _B * _H * _W

_NW = 32
_ROWS = 128
_HB = 32768
_NSTRIPE = 16
_SW = _HB // _NSTRIPE
_TARGET = _MIN_KEPT + 1
_THRESH_BITS = int(np.float32(_THRESH).view(np.int32))

_mesh = plsc.VectorSubcoreMesh(core_axis_name="c", subcore_axis_name="s")


def _lane(v, j):
    return jnp.max(jnp.where(lax.iota(jnp.int32, 16) == j, v, jnp.int32(0)))


def _pass1_body(pred_ref, tgt_ref, prob_ref, nll_ref, s07_ref, acc_ref):
    x = pred_ref[0]
    t = tgt_ref[0]
    tt = jnp.where(t == _IGNORE, 0, t)
    m = jnp.max(x, axis=0)
    e = jnp.exp(x - m[None])
    s = jnp.sum(e, axis=0)
    onehot = jax.lax.broadcasted_iota(jnp.int32, (_C, _TH, _W), 0) == tt[None]
    xt = jnp.sum(jnp.where(onehot, x, 0.0), axis=0)
    prob = jnp.exp(xt - m) / s
    nll = jnp.log(s) + m - xt
    prob_ref[0] = prob
    nll_ref[0] = nll

    step = pl.program_id(0) * (_H // _TH) + pl.program_id(1)

    @pl.when(step == 0)
    def _():
        acc_ref[0] = 0.0
        acc_ref[1] = 0.0

    keep = prob < _THRESH
    acc_ref[0] += jnp.sum(jnp.where(keep, nll, 0.0))
    acc_ref[1] += jnp.sum(keep.astype(jnp.float32))

    @pl.when(step == _B * (_H // _TH) - 1)
    def _():
        s07_ref[0, 0] = acc_ref[0]
        s07_ref[0, 1] = acc_ref[1]


def _hist_round(stage):
    out_type = (
        jax.ShapeDtypeStruct((_NSTRIPE, _NW, _SW), jnp.int32),
        jax.ShapeDtypeStruct((_NW, 16), jnp.int32),
    )
    scratch = [
        pltpu.VMEM((_ROWS // 2, _W), jnp.float32),
        pltpu.VMEM((_ROWS // 2, _W), jnp.float32),
        pltpu.VMEM((_HB,), jnp.int32),
        pltpu.VMEM((16,), jnp.int32),
        pltpu.VMEM((16,), jnp.int32),
        pltpu.SemaphoreType.DMA,
        pltpu.SemaphoreType.DMA,
    ]

    def body(prob_hbm, *rest):
        if stage == 1:
            (hist_out, coarse_out, buf_a, buf_b, hist_v, coarse_v, pick_v,
             sem_a, sem_b) = rest
        else:
            (pick_hbm, hist_out, coarse_out, buf_a, buf_b, hist_v, coarse_v,
             pick_v, sem_a, sem_b) = rest
        wid = lax.axis_index("s") * 2 + lax.axis_index("c")
        bq = lax.shift_right_logical(wid, 2)
        hq = jnp.bitwise_and(wid, 3) * _ROWS
        half = _ROWS // 2
        cp_a = pltpu.async_copy(prob_hbm.at[bq, pl.ds(hq, half)], buf_a, sem_a)
        cp_b = pltpu.async_copy(
            prob_hbm.at[bq, pl.ds(hq + half, half)], buf_b, sem_b)
        if stage == 2:
            pltpu.sync_copy(pick_hbm, pick_v)
            b1 = _lane(pick_v[...], 0)

        zeros = jnp.zeros((16,), jnp.int32)

        def zero_body(i, _):
            hist_v[pl.ds(pl.multiple_of(i * 16, 16), 16)] = zeros
            return 0

        lax.fori_loop(0, _HB // 16, zero_body, 0, unroll=8)

        ones = jnp.ones((16,), jnp.int32)

        buf = buf_a

        def row_body(r, _):
            for j in range(_W // 16):
                v = buf[r, pl.ds(j * 16, 16)]
                bits = lax.bitcast_convert_type(v, jnp.int32)
                if stage == 1:
                    plsc.addupdate_scatter(
                        hist_v, [lax.shift_right_logical(bits, 15)], ones)
                else:
                    match = lax.shift_right_logical(bits, 15) == b1
                    plsc.addupdate_scatter(
                        hist_v, [jnp.bitwise_and(bits, _HB - 1)], ones,
                        mask=match)
            return 0

        cp_a.wait()
        lax.fori_loop(0, _ROWS // 2, row_body, 0)
        buf = buf_b
        cp_b.wait()
        lax.fori_loop(0, _ROWS // 2, row_body, 0)

        coarse = zeros
        lanes = lax.iota(jnp.int32, 16)
        for s in range(_NSTRIPE):
            def sum_body(j, a, s=s):
                return a + hist_v[pl.ds(pl.multiple_of(s * _SW + j * 16, 16),
                                        16)]
            acc = lax.fori_loop(0, _SW // 16, sum_body, zeros, unroll=8)
            coarse = jnp.where(lanes == s, jnp.sum(acc), coarse)
        coarse_v[...] = coarse

        copies = [
            pltpu.async_copy(
                hist_v.at[pl.ds(s * _SW, _SW)], hist_out.at[s, wid], sem_a)
            for s in range(_NSTRIPE)
        ]
        for c in copies:
            c.wait()
        pltpu.sync_copy(coarse_v, coarse_out.at[wid])

    return functools.partial(
        pl.kernel, out_type=out_type, mesh=_mesh, scratch_types=scratch,
        compiler_params=pltpu.CompilerParams(
            needs_layout_passes=False))(body)


_sc_hist1 = _hist_round(1)
_sc_hist2 = _hist_round(2)


def _pick_round(stage):
    scratch = [
        pltpu.VMEM((_NW, _SW), jnp.int32),
        pltpu.VMEM((_NW, 16), jnp.int32),
        pltpu.VMEM((16,), jnp.int32),
        pltpu.VMEM((16,), jnp.int32),
    ]

    def body(hist_hbm, coarse_hbm, *rest):
        if stage == 1:
            (out_hbm, sbuf, cbuf, ovec, pick_v) = rest
        else:
            (pick_hbm, out_hbm, sbuf, cbuf, ovec, pick_v) = rest
        sid = lax.axis_index("s")
        pltpu.sync_copy(coarse_hbm, cbuf)
        if stage == 1:
            target = jnp.int32(_TARGET)
        else:
            pltpu.sync_copy(pick_hbm, pick_v)
            b1 = _lane(pick_v[...], 0)
            target = _lane(pick_v[...], 1)

        tot = jnp.zeros((16,), jnp.int32)
        for r in range(_NW):
            tot = tot + cbuf[r]
        cum = plsc.cumsum(tot)
        sstar = jnp.max(plsc.all_reduce_ffs(cum >= target))
        before_stripe = jnp.max(jnp.where(cum < target, cum, jnp.int32(0)))
        krem = target - before_stripe

        pltpu.sync_copy(hist_hbm.at[sid], sbuf)

        def chunk(c, carry):
            bin_, before, run = carry
            acc = jnp.zeros((16,), jnp.int32)
            for r in range(_NW):
                acc = acc + sbuf[r, pl.ds(pl.multiple_of(c * 16, 16), 16)]
            cs = plsc.cumsum(acc) + run
            ge = cs >= krem
            has = jnp.max(ge.astype(jnp.int32)) > 0
            idx = jnp.max(plsc.all_reduce_ffs(ge))
            fresh = (bin_ < 0) & has
            newbin = jnp.where(fresh, c * 16 + idx, bin_)
            newbefore = jnp.where(
                fresh, jnp.max(jnp.where(cs < krem, cs, run)), before)
            return (newbin, newbefore, jnp.max(cs))

        bin_, before, _ = lax.fori_loop(
            0, _SW // 16, chunk, (jnp.int32(-1), jnp.int32(0), jnp.int32(0)))

        lanes = lax.iota(jnp.int32, 16)
        if stage == 1:
            bg = sid * _SW + bin_
            krem2 = krem - before
            pred = (bg < jnp.int32(_THRESH_BITS >> 15)).astype(jnp.int32)
            vec = jnp.where(lanes == 0, bg,
                            jnp.where(lanes == 1, krem2,
                                      jnp.where(lanes == 2, pred,
                                                jnp.int32(0))))
        else:
            t = jnp.bitwise_or(lax.shift_left(b1, 15), sid * _SW + bin_)
            vec = jnp.zeros((16,), jnp.int32) + jnp.maximum(
                t, jnp.int32(_THRESH_BITS))
        ovec[...] = vec

        @pl.when(sid == sstar)
        def _():
            pltpu.sync_copy(ovec, out_hbm)

    return functools.partial(
        pl.kernel, out_type=jax.ShapeDtypeStruct((16,), jnp.int32),
        mesh=_mesh, scratch_types=scratch,
        compiler_params=pltpu.CompilerParams(
            needs_layout_passes=False))(body)


_sc_pick1 = _pick_round(1)
_sc_pick2 = _pick_round(2)


def _final_body(thr_ref, prob_ref, nll_ref, out_ref):
    thr = jax.lax.bitcast_convert_type(thr_ref[0, 0], jnp.float32)
    prob = prob_ref[...]
    keep = prob < thr
    s = jnp.sum(jnp.where(keep, nll_ref[...], 0.0))
    c = jnp.sum(keep.astype(jnp.float32))
    out_ref[0, 0] = s / c


def kernel(predict, target):
    tgt = target.astype(jnp.int32)

    prob, nll, s07 = pl.pallas_call(
        _pass1_body,
        grid=(_B, _H // _TH),
        in_specs=[
            pl.BlockSpec((1, _C, _TH, _W), lambda b, h: (b, 0, h, 0)),
            pl.BlockSpec((1, _TH, _W), lambda b, h: (b, h, 0)),
        ],
        out_specs=[
            pl.BlockSpec((1, _TH, _W), lambda b, h: (b, h, 0)),
            pl.BlockSpec((1, _TH, _W), lambda b, h: (b, h, 0)),
            pl.BlockSpec(memory_space=pltpu.SMEM),
        ],
        out_shape=[
            jax.ShapeDtypeStruct((_B, _H, _W), jnp.float32),
            jax.ShapeDtypeStruct((_B, _H, _W), jnp.float32),
            jax.ShapeDtypeStruct((1, 2), jnp.float32),
        ],
        scratch_shapes=[pltpu.SMEM((2,), jnp.float32)],
    )(predict, tgt)

    hist1, coarse1 = _sc_hist1(prob)
    pick1 = _sc_pick1(hist1, coarse1)

    def _common(_):
        return s07[0, 0] / s07[0, 1]

    def _rare(_):
        hist2, coarse2 = _sc_hist2(prob, pick1)
        thr_vec = _sc_pick2(hist2, coarse2, pick1)
        loss = pl.pallas_call(
            _final_body,
            in_specs=[
                pl.BlockSpec(memory_space=pltpu.SMEM),
                pl.BlockSpec((_B, _H, _W), lambda: (0, 0, 0)),
                pl.BlockSpec((_B, _H, _W), lambda: (0, 0, 0)),
            ],
            out_specs=pl.BlockSpec(memory_space=pltpu.SMEM),
            out_shape=jax.ShapeDtypeStruct((1, 1), jnp.float32),
        )(thr_vec[0:1].reshape(1, 1), prob, nll)
        return loss[0, 0]

    return lax.cond(pick1[2] == 1, _common, _rare, None)
